# baseline (device time: 136200 ns/iter reference)
import jax
import jax.numpy as jnp
from jax import lax
from jax.experimental import pallas as pl
from jax.experimental.pallas import tpu as pltpu

N_DEV = 8
N_TOK = 2048
D_MODEL = 512
D_HID = 1024
N_EXP = 32
EXP_PER_DEV = N_EXP // N_DEV
CHUNK = N_TOK // N_DEV
HALF = CHUNK // 2
RS_HOPS = N_DEV - 1
AG_HOPS = N_DEV - 1
N_SEM = RS_HOPS + AG_HOPS


def _mod(v, n):
    return lax.rem(v + 4 * n, n)


def kernel(x, router_W, route_idx, expert_W):
    def body(x_ref, rw_ref, idx_ref, ew_ref, out_ref,
             comm_p, comm_m, send_p, recv_p, send_m, recv_m):
        my = lax.axis_index("i")
        left = _mod(my - 1, N_DEV)
        right = _mod(my + 1, N_DEV)

        barrier = pltpu.get_barrier_semaphore()
        for nbr in (left, right):
            pl.semaphore_signal(barrier, inc=1, device_id=(nbr,),
                                device_id_type=pl.DeviceIdType.MESH)
        pl.semaphore_wait(barrier, 2)

        xall = x_ref[:, :]
        scores = jnp.dot(xall, rw_ref[:, :],
                         preferred_element_type=jnp.float32)
        smax = jnp.max(scores, axis=-1, keepdims=True)
        probs = jnp.exp(scores - smax)
        probs = probs / jnp.sum(probs, axis=-1, keepdims=True)
        e0 = idx_ref[:, 0:1]
        e1 = idx_ref[:, 1:2]
        col = lax.broadcasted_iota(jnp.int32, (N_TOK, N_EXP), 1)
        g0 = jnp.sum(jnp.where(col == e0, probs, 0.0), axis=-1, keepdims=True)
        g1 = jnp.sum(jnp.where(col == e1, probs, 0.0), axis=-1, keepdims=True)
        gsum = g0 + g1
        g0 = g0 / gsum
        g1 = g1 / gsum

        base = my * EXP_PER_DEV
        for c in range(N_DEV):
            rows = slice(c * CHUNK, (c + 1) * CHUNK)
            xc = xall[rows, :]
            acc = jnp.zeros((CHUNK, D_HID), jnp.float32)
            for k in range(EXP_PER_DEV):
                ge = base + k
                gate = (jnp.where(e0[rows] == ge, g0[rows], 0.0)
                        + jnp.where(e1[rows] == ge, g1[rows], 0.0))
                acc = acc + jnp.dot(xc * gate, ew_ref[k],
                                    preferred_element_type=jnp.float32)
            out_ref[rows, :] = acc

        for s in range(RS_HOPS):
            cp = _mod(my - s, N_DEV)
            cm = _mod(my + s, N_DEV)
            rp = pltpu.make_async_remote_copy(
                src_ref=out_ref.at[pl.ds(cp * CHUNK, HALF), :],
                dst_ref=comm_p.at[s],
                send_sem=send_p.at[s],
                recv_sem=recv_p.at[s],
                device_id=(right,),
                device_id_type=pl.DeviceIdType.MESH,
            )
            rm = pltpu.make_async_remote_copy(
                src_ref=out_ref.at[pl.ds(cm * CHUNK + HALF, HALF), :],
                dst_ref=comm_m.at[s],
                send_sem=send_m.at[s],
                recv_sem=recv_m.at[s],
                device_id=(left,),
                device_id_type=pl.DeviceIdType.MESH,
            )
            rp.start()
            rm.start()
            rp.wait()
            rm.wait()
            ap = _mod(my - s - 1, N_DEV)
            am = _mod(my + s + 1, N_DEV)
            slp = pl.ds(ap * CHUNK, HALF)
            slm = pl.ds(am * CHUNK + HALF, HALF)
            out_ref[slp, :] = out_ref[slp, :] + comm_p[s, :, :]
            out_ref[slm, :] = out_ref[slm, :] + comm_m[s, :, :]

        for s in range(AG_HOPS):
            cp = _mod(my + 1 - s, N_DEV)
            cm = _mod(my - 1 + s, N_DEV)
            slp = pl.ds(cp * CHUNK, HALF)
            slm = pl.ds(cm * CHUNK + HALF, HALF)
            rp = pltpu.make_async_remote_copy(
                src_ref=out_ref.at[slp, :],
                dst_ref=out_ref.at[slp, :],
                send_sem=send_p.at[RS_HOPS + s],
                recv_sem=recv_p.at[RS_HOPS + s],
                device_id=(right,),
                device_id_type=pl.DeviceIdType.MESH,
            )
            rm = pltpu.make_async_remote_copy(
                src_ref=out_ref.at[slm, :],
                dst_ref=out_ref.at[slm, :],
                send_sem=send_m.at[RS_HOPS + s],
                recv_sem=recv_m.at[RS_HOPS + s],
                device_id=(left,),
                device_id_type=pl.DeviceIdType.MESH,
            )
            rp.start()
            rm.start()
            rp.wait()
            rm.wait()

    return pl.pallas_call(
        body,
        out_shape=jax.ShapeDtypeStruct((N_TOK, D_HID), jnp.float32),
        in_specs=[
            pl.BlockSpec(memory_space=pltpu.VMEM),
            pl.BlockSpec(memory_space=pltpu.VMEM),
            pl.BlockSpec(memory_space=pltpu.VMEM),
            pl.BlockSpec(memory_space=pltpu.VMEM),
        ],
        out_specs=pl.BlockSpec(memory_space=pltpu.VMEM),
        scratch_shapes=[
            pltpu.VMEM((RS_HOPS, HALF, D_HID), jnp.float32),
            pltpu.VMEM((RS_HOPS, HALF, D_HID), jnp.float32),
            pltpu.SemaphoreType.DMA((N_SEM,)),
            pltpu.SemaphoreType.DMA((N_SEM,)),
            pltpu.SemaphoreType.DMA((N_SEM,)),
            pltpu.SemaphoreType.DMA((N_SEM,)),
        ],
        compiler_params=pltpu.CompilerParams(collective_id=0),
    )(x, router_W, route_idx, expert_W)


# device time: 98093 ns/iter; 1.3885x vs baseline; 1.3885x over previous
import jax
import jax.numpy as jnp
from jax import lax
from jax.experimental import pallas as pl
from jax.experimental.pallas import tpu as pltpu

N_DEV = 8
N_TOK = 2048
D_MODEL = 512
D_HID = 1024
N_EXP = 32
EXP_PER_DEV = N_EXP // N_DEV
CHUNK = N_TOK // N_DEV
HALF = CHUNK // 2
RS_HOPS = N_DEV - 1
AG_HOPS = N_DEV - 1
N_SEM = RS_HOPS + AG_HOPS


def _mod(v, n):
    return lax.rem(v + 4 * n, n)


def kernel(x, router_W, route_idx, expert_W):
    def body(x_ref, rw_ref, idx_ref, ew_ref, out_ref,
             comm_p, comm_m, stage_p, stage_m, ag_p, ag_m,
             send_p, recv_p, send_m, recv_m):
        my = lax.axis_index("i")
        left = _mod(my - 1, N_DEV)
        right = _mod(my + 1, N_DEV)

        barrier = pltpu.get_barrier_semaphore()
        for nbr in (left, right):
            pl.semaphore_signal(barrier, inc=1, device_id=(nbr,),
                                device_id_type=pl.DeviceIdType.MESH)
        pl.semaphore_wait(barrier, 2)

        xall = x_ref[:, :]
        scores = jnp.dot(xall, rw_ref[:, :],
                         preferred_element_type=jnp.float32)
        smax = jnp.max(scores, axis=-1, keepdims=True)
        probs = jnp.exp(scores - smax)
        probs = probs / jnp.sum(probs, axis=-1, keepdims=True)
        e0 = idx_ref[:, 0:1]
        e1 = idx_ref[:, 1:2]
        col = lax.broadcasted_iota(jnp.int32, (N_TOK, N_EXP), 1)
        g0 = jnp.sum(jnp.where(col == e0, probs, 0.0), axis=-1, keepdims=True)
        g1 = jnp.sum(jnp.where(col == e1, probs, 0.0), axis=-1, keepdims=True)
        gsum = g0 + g1
        g0 = g0 / gsum
        g1 = g1 / gsum

        base = my * EXP_PER_DEV
        for c in range(N_DEV):
            rows = slice(c * CHUNK, (c + 1) * CHUNK)
            xc = xall[rows, :]
            acc = jnp.zeros((CHUNK, D_HID), jnp.float32)
            for k in range(EXP_PER_DEV):
                ge = base + k
                gate = (jnp.where(e0[rows] == ge, g0[rows], 0.0)
                        + jnp.where(e1[rows] == ge, g1[rows], 0.0))
                acc = acc + jnp.dot(xc * gate, ew_ref[k],
                                    preferred_element_type=jnp.float32)
            out_ref[rows, :] = acc

        stage_p[0, :, :] = out_ref[pl.ds(_mod(my, N_DEV) * CHUNK, HALF), :].astype(jnp.bfloat16)
        stage_m[0, :, :] = out_ref[pl.ds(_mod(my, N_DEV) * CHUNK + HALF, HALF), :].astype(jnp.bfloat16)
        for s in range(RS_HOPS):
            rp = pltpu.make_async_remote_copy(
                src_ref=stage_p.at[s],
                dst_ref=comm_p.at[s],
                send_sem=send_p.at[s],
                recv_sem=recv_p.at[s],
                device_id=(right,),
                device_id_type=pl.DeviceIdType.MESH,
            )
            rm = pltpu.make_async_remote_copy(
                src_ref=stage_m.at[s],
                dst_ref=comm_m.at[s],
                send_sem=send_m.at[s],
                recv_sem=recv_m.at[s],
                device_id=(left,),
                device_id_type=pl.DeviceIdType.MESH,
            )
            rp.start()
            rm.start()
            rp.wait()
            rm.wait()
            ap = _mod(my - s - 1, N_DEV)
            am = _mod(my + s + 1, N_DEV)
            slp = pl.ds(ap * CHUNK, HALF)
            slm = pl.ds(am * CHUNK + HALF, HALF)
            vp = out_ref[slp, :] + comm_p[s, :, :].astype(jnp.float32)
            vm = out_ref[slm, :] + comm_m[s, :, :].astype(jnp.float32)
            out_ref[slp, :] = vp
            out_ref[slm, :] = vm
            if s < RS_HOPS - 1:
                stage_p[s + 1, :, :] = vp.astype(jnp.bfloat16)
                stage_m[s + 1, :, :] = vm.astype(jnp.bfloat16)
            else:
                ag_p[0, :, :] = vp.astype(jnp.bfloat16)
                ag_m[0, :, :] = vm.astype(jnp.bfloat16)

        for s in range(AG_HOPS):
            rp = pltpu.make_async_remote_copy(
                src_ref=ag_p.at[s],
                dst_ref=ag_p.at[s + 1],
                send_sem=send_p.at[RS_HOPS + s],
                recv_sem=recv_p.at[RS_HOPS + s],
                device_id=(right,),
                device_id_type=pl.DeviceIdType.MESH,
            )
            rm = pltpu.make_async_remote_copy(
                src_ref=ag_m.at[s],
                dst_ref=ag_m.at[s + 1],
                send_sem=send_m.at[RS_HOPS + s],
                recv_sem=recv_m.at[RS_HOPS + s],
                device_id=(left,),
                device_id_type=pl.DeviceIdType.MESH,
            )
            rp.start()
            rm.start()
            rp.wait()
            rm.wait()
            cp = _mod(my - s, N_DEV)
            cm = _mod(my + s, N_DEV)
            out_ref[pl.ds(cp * CHUNK, HALF), :] = ag_p[s + 1, :, :].astype(jnp.float32)
            out_ref[pl.ds(cm * CHUNK + HALF, HALF), :] = ag_m[s + 1, :, :].astype(jnp.float32)

    return pl.pallas_call(
        body,
        out_shape=jax.ShapeDtypeStruct((N_TOK, D_HID), jnp.float32),
        in_specs=[
            pl.BlockSpec(memory_space=pltpu.VMEM),
            pl.BlockSpec(memory_space=pltpu.VMEM),
            pl.BlockSpec(memory_space=pltpu.VMEM),
            pl.BlockSpec(memory_space=pltpu.VMEM),
        ],
        out_specs=pl.BlockSpec(memory_space=pltpu.VMEM),
        scratch_shapes=[
            pltpu.VMEM((RS_HOPS, HALF, D_HID), jnp.bfloat16),
            pltpu.VMEM((RS_HOPS, HALF, D_HID), jnp.bfloat16),
            pltpu.VMEM((RS_HOPS, HALF, D_HID), jnp.bfloat16),
            pltpu.VMEM((RS_HOPS, HALF, D_HID), jnp.bfloat16),
            pltpu.VMEM((N_DEV, HALF, D_HID), jnp.bfloat16),
            pltpu.VMEM((N_DEV, HALF, D_HID), jnp.bfloat16),
            pltpu.SemaphoreType.DMA((N_SEM,)),
            pltpu.SemaphoreType.DMA((N_SEM,)),
            pltpu.SemaphoreType.DMA((N_SEM,)),
            pltpu.SemaphoreType.DMA((N_SEM,)),
        ],
        compiler_params=pltpu.CompilerParams(collective_id=0),
    )(x, router_W, route_idx, expert_W)


# device time: 89537 ns/iter; 1.5212x vs baseline; 1.0956x over previous
import jax
import jax.numpy as jnp
from jax import lax
from jax.experimental import pallas as pl
from jax.experimental.pallas import tpu as pltpu

N_DEV = 8
N_TOK = 2048
D_MODEL = 512
D_HID = 1024
N_EXP = 32
EXP_PER_DEV = N_EXP // N_DEV
CHUNK = N_TOK // N_DEV
HALF = CHUNK // 2
RS_HOPS = N_DEV - 1
AG_HOPS = N_DEV - 1
N_SEM = RS_HOPS + AG_HOPS


def _mod(v, n):
    return lax.rem(v + 4 * n, n)


def kernel(x, router_W, route_idx, expert_W):
    def body(x_ref, rw_ref, idx_ref, ew_ref, out_ref,
             gb_ref, comm_p, comm_m, stage_p, stage_m, ag_p, ag_m,
             send_p, recv_p, send_m, recv_m):
        my = lax.axis_index("i")
        left = _mod(my - 1, N_DEV)
        right = _mod(my + 1, N_DEV)

        barrier = pltpu.get_barrier_semaphore()
        for nbr in (left, right):
            pl.semaphore_signal(barrier, inc=1, device_id=(nbr,),
                                device_id_type=pl.DeviceIdType.MESH)
        pl.semaphore_wait(barrier, 2)

        xall = x_ref[:, :]
        scores = jnp.dot(xall, rw_ref[:, :],
                         preferred_element_type=jnp.float32)
        smax = jnp.max(scores, axis=-1, keepdims=True)
        probs = jnp.exp(scores - smax)
        probs = probs / jnp.sum(probs, axis=-1, keepdims=True)
        e0 = idx_ref[:, 0:1]
        e1 = idx_ref[:, 1:2]
        col = lax.broadcasted_iota(jnp.int32, (N_TOK, N_EXP), 1)
        g0 = jnp.sum(jnp.where(col == e0, probs, 0.0), axis=-1, keepdims=True)
        g1 = jnp.sum(jnp.where(col == e1, probs, 0.0), axis=-1, keepdims=True)
        gsum = g0 + g1
        gb_ref[:, 0:1] = g0 / gsum
        gb_ref[:, 1:2] = g1 / gsum

        base = my * EXP_PER_DEV

        def compute_chunk(off):
            xc = x_ref[pl.ds(off, CHUNK), :]
            e0c = idx_ref[pl.ds(off, CHUNK), 0:1]
            e1c = idx_ref[pl.ds(off, CHUNK), 1:2]
            g0c = gb_ref[pl.ds(off, CHUNK), 0:1]
            g1c = gb_ref[pl.ds(off, CHUNK), 1:2]
            acc = jnp.zeros((CHUNK, D_HID), jnp.float32)
            for k in range(EXP_PER_DEV):
                ge = base + k
                gate = (jnp.where(e0c == ge, g0c, 0.0)
                        + jnp.where(e1c == ge, g1c, 0.0))
                acc = acc + jnp.dot(xc * gate, ew_ref[k],
                                    preferred_element_type=jnp.float32)
            return acc

        _D_ORDER = [0, -1, 1, -2, 2, -3, 3, 4]

        def off_of(j):
            return _mod(my + _D_ORDER[j], N_DEV) * CHUNK

        def rs_copies(s):
            rp = pltpu.make_async_remote_copy(
                src_ref=stage_p.at[s],
                dst_ref=comm_p.at[s],
                send_sem=send_p.at[s],
                recv_sem=recv_p.at[s],
                device_id=(right,),
                device_id_type=pl.DeviceIdType.MESH,
            )
            rm = pltpu.make_async_remote_copy(
                src_ref=stage_m.at[s],
                dst_ref=comm_m.at[s],
                send_sem=send_m.at[s],
                recv_sem=recv_m.at[s],
                device_id=(left,),
                device_id_type=pl.DeviceIdType.MESH,
            )
            return rp, rm

        def ag_copies(s):
            rp = pltpu.make_async_remote_copy(
                src_ref=ag_p.at[s],
                dst_ref=ag_p.at[s + 1],
                send_sem=send_p.at[RS_HOPS + s],
                recv_sem=recv_p.at[RS_HOPS + s],
                device_id=(right,),
                device_id_type=pl.DeviceIdType.MESH,
            )
            rm = pltpu.make_async_remote_copy(
                src_ref=ag_m.at[s],
                dst_ref=ag_m.at[s + 1],
                send_sem=send_m.at[RS_HOPS + s],
                recv_sem=recv_m.at[RS_HOPS + s],
                device_id=(left,),
                device_id_type=pl.DeviceIdType.MESH,
            )
            return rp, rm

        acc0 = compute_chunk(off_of(0))
        stage_p[0, :, :] = acc0[:HALF, :].astype(jnp.bfloat16)
        stage_m[0, :, :] = acc0[HALF:, :].astype(jnp.bfloat16)
        rp, rm = rs_copies(0)
        rp.start()
        rm.start()
        out_ref[pl.ds(off_of(0), CHUNK), :] = acc0

        next_j = 1
        for s in range(RS_HOPS):
            for _ in range(2):
                if next_j < N_DEV:
                    off = off_of(next_j)
                    out_ref[pl.ds(off, CHUNK), :] = compute_chunk(off)
                    next_j += 1
            rp.wait()
            rm.wait()
            ap = _mod(my - s - 1, N_DEV)
            am = _mod(my + s + 1, N_DEV)
            slp = pl.ds(ap * CHUNK, HALF)
            slm = pl.ds(am * CHUNK + HALF, HALF)
            vp = out_ref[slp, :] + comm_p[s, :, :].astype(jnp.float32)
            vm = out_ref[slm, :] + comm_m[s, :, :].astype(jnp.float32)
            if s < RS_HOPS - 1:
                stage_p[s + 1, :, :] = vp.astype(jnp.bfloat16)
                stage_m[s + 1, :, :] = vm.astype(jnp.bfloat16)
                rp, rm = rs_copies(s + 1)
            else:
                ag_p[0, :, :] = vp.astype(jnp.bfloat16)
                ag_m[0, :, :] = vm.astype(jnp.bfloat16)
                rp, rm = ag_copies(0)
            rp.start()
            rm.start()
            out_ref[slp, :] = vp
            out_ref[slm, :] = vm

        for s in range(AG_HOPS):
            rp.wait()
            rm.wait()
            if s < AG_HOPS - 1:
                rp, rm = ag_copies(s + 1)
                rp.start()
                rm.start()
            cp = _mod(my - s, N_DEV)
            cm = _mod(my + s, N_DEV)
            out_ref[pl.ds(cp * CHUNK, HALF), :] = ag_p[s + 1, :, :].astype(jnp.float32)
            out_ref[pl.ds(cm * CHUNK + HALF, HALF), :] = ag_m[s + 1, :, :].astype(jnp.float32)

    return pl.pallas_call(
        body,
        out_shape=jax.ShapeDtypeStruct((N_TOK, D_HID), jnp.float32),
        in_specs=[
            pl.BlockSpec(memory_space=pltpu.VMEM),
            pl.BlockSpec(memory_space=pltpu.VMEM),
            pl.BlockSpec(memory_space=pltpu.VMEM),
            pl.BlockSpec(memory_space=pltpu.VMEM),
        ],
        out_specs=pl.BlockSpec(memory_space=pltpu.VMEM),
        scratch_shapes=[
            pltpu.VMEM((N_TOK, 2), jnp.float32),
            pltpu.VMEM((RS_HOPS, HALF, D_HID), jnp.bfloat16),
            pltpu.VMEM((RS_HOPS, HALF, D_HID), jnp.bfloat16),
            pltpu.VMEM((RS_HOPS, HALF, D_HID), jnp.bfloat16),
            pltpu.VMEM((RS_HOPS, HALF, D_HID), jnp.bfloat16),
            pltpu.VMEM((N_DEV, HALF, D_HID), jnp.bfloat16),
            pltpu.VMEM((N_DEV, HALF, D_HID), jnp.bfloat16),
            pltpu.SemaphoreType.DMA((N_SEM,)),
            pltpu.SemaphoreType.DMA((N_SEM,)),
            pltpu.SemaphoreType.DMA((N_SEM,)),
            pltpu.SemaphoreType.DMA((N_SEM,)),
        ],
        compiler_params=pltpu.CompilerParams(collective_id=0),
    )(x, router_W, route_idx, expert_W)


# device time: 84312 ns/iter; 1.6154x vs baseline; 1.0620x over previous
import jax
import jax.numpy as jnp
from jax import lax
from jax.experimental import pallas as pl
from jax.experimental.pallas import tpu as pltpu

N_DEV = 8
N_TOK = 2048
D_MODEL = 512
D_HID = 1024
N_EXP = 32
EXP_PER_DEV = N_EXP // N_DEV
CHUNK = N_TOK // N_DEV
HALF = CHUNK // 2
RS_HOPS = N_DEV - 1
AG_HOPS = N_DEV - 1
N_SEM = RS_HOPS + AG_HOPS


def _mod(v, n):
    return lax.rem(v + 4 * n, n)


def kernel(x, router_W, route_idx, expert_W):
    def body(x_ref, rw_ref, idx_ref, ew_ref, out_ref,
             gb_ref, comm_p, comm_m, stage_p, stage_m, ag_p, ag_m,
             send_p, recv_p, send_m, recv_m):
        my = lax.axis_index("i")
        def _perm(q):
            return jnp.where(q < 4, q, 11 - q)

        r = _perm(my)
        left = _perm(_mod(r - 1, N_DEV))
        right = _perm(_mod(r + 1, N_DEV))

        barrier = pltpu.get_barrier_semaphore()
        for nbr in (left, right):
            pl.semaphore_signal(barrier, inc=1, device_id=(nbr,),
                                device_id_type=pl.DeviceIdType.MESH)
        pl.semaphore_wait(barrier, 2)

        xall = x_ref[:, :]
        scores = jnp.dot(xall, rw_ref[:, :],
                         preferred_element_type=jnp.float32)
        smax = jnp.max(scores, axis=-1, keepdims=True)
        probs = jnp.exp(scores - smax)
        probs = probs / jnp.sum(probs, axis=-1, keepdims=True)
        e0 = idx_ref[:, 0:1]
        e1 = idx_ref[:, 1:2]
        col = lax.broadcasted_iota(jnp.int32, (N_TOK, N_EXP), 1)
        g0 = jnp.sum(jnp.where(col == e0, probs, 0.0), axis=-1, keepdims=True)
        g1 = jnp.sum(jnp.where(col == e1, probs, 0.0), axis=-1, keepdims=True)
        gsum = g0 + g1
        gb_ref[:, 0:1] = g0 / gsum
        gb_ref[:, 1:2] = g1 / gsum

        base = my * EXP_PER_DEV

        def compute_chunk(off):
            xc = x_ref[pl.ds(off, CHUNK), :]
            e0c = idx_ref[pl.ds(off, CHUNK), 0:1]
            e1c = idx_ref[pl.ds(off, CHUNK), 1:2]
            g0c = gb_ref[pl.ds(off, CHUNK), 0:1]
            g1c = gb_ref[pl.ds(off, CHUNK), 1:2]
            acc = jnp.zeros((CHUNK, D_HID), jnp.float32)
            for k in range(EXP_PER_DEV):
                ge = base + k
                gate = (jnp.where(e0c == ge, g0c, 0.0)
                        + jnp.where(e1c == ge, g1c, 0.0))
                acc = acc + jnp.dot(xc * gate, ew_ref[k],
                                    preferred_element_type=jnp.float32)
            return acc

        _D_ORDER = [0, -1, 1, -2, 2, -3, 3, 4]

        def off_of(j):
            return _perm(_mod(r + _D_ORDER[j], N_DEV)) * CHUNK

        def rs_copies(s):
            rp = pltpu.make_async_remote_copy(
                src_ref=stage_p.at[s],
                dst_ref=comm_p.at[s],
                send_sem=send_p.at[s],
                recv_sem=recv_p.at[s],
                device_id=(right,),
                device_id_type=pl.DeviceIdType.MESH,
            )
            rm = pltpu.make_async_remote_copy(
                src_ref=stage_m.at[s],
                dst_ref=comm_m.at[s],
                send_sem=send_m.at[s],
                recv_sem=recv_m.at[s],
                device_id=(left,),
                device_id_type=pl.DeviceIdType.MESH,
            )
            return rp, rm

        def ag_copies(s):
            rp = pltpu.make_async_remote_copy(
                src_ref=ag_p.at[s],
                dst_ref=ag_p.at[s + 1],
                send_sem=send_p.at[RS_HOPS + s],
                recv_sem=recv_p.at[RS_HOPS + s],
                device_id=(right,),
                device_id_type=pl.DeviceIdType.MESH,
            )
            rm = pltpu.make_async_remote_copy(
                src_ref=ag_m.at[s],
                dst_ref=ag_m.at[s + 1],
                send_sem=send_m.at[RS_HOPS + s],
                recv_sem=recv_m.at[RS_HOPS + s],
                device_id=(left,),
                device_id_type=pl.DeviceIdType.MESH,
            )
            return rp, rm

        acc0 = compute_chunk(off_of(0))
        stage_p[0, :, :] = acc0[:HALF, :].astype(jnp.bfloat16)
        stage_m[0, :, :] = acc0[HALF:, :].astype(jnp.bfloat16)
        rp, rm = rs_copies(0)
        rp.start()
        rm.start()
        out_ref[pl.ds(off_of(0), CHUNK), :] = acc0

        next_j = 1
        for s in range(RS_HOPS):
            for _ in range(2):
                if next_j < N_DEV:
                    off = off_of(next_j)
                    out_ref[pl.ds(off, CHUNK), :] = compute_chunk(off)
                    next_j += 1
            rp.wait()
            rm.wait()
            ap = _perm(_mod(r - s - 1, N_DEV))
            am = _perm(_mod(r + s + 1, N_DEV))
            slp = pl.ds(ap * CHUNK, HALF)
            slm = pl.ds(am * CHUNK + HALF, HALF)
            vp = out_ref[slp, :] + comm_p[s, :, :].astype(jnp.float32)
            vm = out_ref[slm, :] + comm_m[s, :, :].astype(jnp.float32)
            if s < RS_HOPS - 1:
                stage_p[s + 1, :, :] = vp.astype(jnp.bfloat16)
                stage_m[s + 1, :, :] = vm.astype(jnp.bfloat16)
                rp, rm = rs_copies(s + 1)
            else:
                ag_p[0, :, :] = vp.astype(jnp.bfloat16)
                ag_m[0, :, :] = vm.astype(jnp.bfloat16)
                rp, rm = ag_copies(0)
            rp.start()
            rm.start()
            out_ref[slp, :] = vp
            out_ref[slm, :] = vm

        for s in range(AG_HOPS):
            rp.wait()
            rm.wait()
            if s < AG_HOPS - 1:
                rp, rm = ag_copies(s + 1)
                rp.start()
                rm.start()
            cp = _perm(_mod(r - s, N_DEV))
            cm = _perm(_mod(r + s, N_DEV))
            out_ref[pl.ds(cp * CHUNK, HALF), :] = ag_p[s + 1, :, :].astype(jnp.float32)
            out_ref[pl.ds(cm * CHUNK + HALF, HALF), :] = ag_m[s + 1, :, :].astype(jnp.float32)

    return pl.pallas_call(
        body,
        out_shape=jax.ShapeDtypeStruct((N_TOK, D_HID), jnp.float32),
        in_specs=[
            pl.BlockSpec(memory_space=pltpu.VMEM),
            pl.BlockSpec(memory_space=pltpu.VMEM),
            pl.BlockSpec(memory_space=pltpu.VMEM),
            pl.BlockSpec(memory_space=pltpu.VMEM),
        ],
        out_specs=pl.BlockSpec(memory_space=pltpu.VMEM),
        scratch_shapes=[
            pltpu.VMEM((N_TOK, 2), jnp.float32),
            pltpu.VMEM((RS_HOPS, HALF, D_HID), jnp.bfloat16),
            pltpu.VMEM((RS_HOPS, HALF, D_HID), jnp.bfloat16),
            pltpu.VMEM((RS_HOPS, HALF, D_HID), jnp.bfloat16),
            pltpu.VMEM((RS_HOPS, HALF, D_HID), jnp.bfloat16),
            pltpu.VMEM((N_DEV, HALF, D_HID), jnp.bfloat16),
            pltpu.VMEM((N_DEV, HALF, D_HID), jnp.bfloat16),
            pltpu.SemaphoreType.DMA((N_SEM,)),
            pltpu.SemaphoreType.DMA((N_SEM,)),
            pltpu.SemaphoreType.DMA((N_SEM,)),
            pltpu.SemaphoreType.DMA((N_SEM,)),
        ],
        compiler_params=pltpu.CompilerParams(collective_id=0),
    )(x, router_W, route_idx, expert_W)


# device time: 83559 ns/iter; 1.6300x vs baseline; 1.0090x over previous
import jax
import jax.numpy as jnp
from jax import lax
from jax.experimental import pallas as pl
from jax.experimental.pallas import tpu as pltpu

N_DEV = 8
N_TOK = 2048
D_MODEL = 512
D_HID = 1024
N_EXP = 32
EXP_PER_DEV = N_EXP // N_DEV
CHUNK = N_TOK // N_DEV
HALF = CHUNK // 2
RS_HOPS = N_DEV - 1
AG_HOPS = N_DEV - 1
N_SEM = RS_HOPS + AG_HOPS


def _mod(v, n):
    return lax.rem(v + 4 * n, n)


def kernel(x, router_W, route_idx, expert_W):
    def body(x_ref, rw_ref, idx_ref, ew_ref, out_ref,
             gb_ref, comm_p, comm_m, stage_p, stage_m, ag_p, ag_m,
             send_p, recv_p, send_m, recv_m):
        my = lax.axis_index("i")
        def _perm(q):
            return jnp.where(q < 4, q, 11 - q)

        r = _perm(my)
        left = _perm(_mod(r - 1, N_DEV))
        right = _perm(_mod(r + 1, N_DEV))

        barrier = pltpu.get_barrier_semaphore()
        for nbr in (left, right):
            pl.semaphore_signal(barrier, inc=1, device_id=(nbr,),
                                device_id_type=pl.DeviceIdType.MESH)
        pl.semaphore_wait(barrier, 2)

        xall = x_ref[:, :]
        scores = jnp.dot(xall, rw_ref[:, :],
                         preferred_element_type=jnp.float32)
        smax = jnp.max(scores, axis=-1, keepdims=True)
        probs = jnp.exp(scores - smax)
        probs = probs / jnp.sum(probs, axis=-1, keepdims=True)
        e0 = idx_ref[:, 0:1]
        e1 = idx_ref[:, 1:2]
        col = lax.broadcasted_iota(jnp.int32, (N_TOK, N_EXP), 1)
        g0 = jnp.sum(jnp.where(col == e0, probs, 0.0), axis=-1, keepdims=True)
        g1 = jnp.sum(jnp.where(col == e1, probs, 0.0), axis=-1, keepdims=True)
        gsum = g0 + g1
        gb_ref[:, 0:1] = g0 / gsum
        gb_ref[:, 1:2] = g1 / gsum

        base = my * EXP_PER_DEV

        def compute_half(off):
            xc = x_ref[pl.ds(off, HALF), :]
            e0c = idx_ref[pl.ds(off, HALF), 0:1]
            e1c = idx_ref[pl.ds(off, HALF), 1:2]
            g0c = gb_ref[pl.ds(off, HALF), 0:1]
            g1c = gb_ref[pl.ds(off, HALF), 1:2]
            acc = jnp.zeros((HALF, D_HID), jnp.float32)
            for k in range(EXP_PER_DEV):
                ge = base + k
                gate = (jnp.where(e0c == ge, g0c, 0.0)
                        + jnp.where(e1c == ge, g1c, 0.0))
                acc = acc + jnp.dot(xc * gate, ew_ref[k],
                                    preferred_element_type=jnp.float32)
            return acc

        def offA(slot):
            return _perm(_mod(slot, N_DEV)) * CHUNK

        def offB(slot):
            return _perm(_mod(slot, N_DEV)) * CHUNK + HALF

        def rs_copies(s):
            rp = pltpu.make_async_remote_copy(
                src_ref=stage_p.at[s],
                dst_ref=comm_p.at[s],
                send_sem=send_p.at[s],
                recv_sem=recv_p.at[s],
                device_id=(right,),
                device_id_type=pl.DeviceIdType.MESH,
            )
            rm = pltpu.make_async_remote_copy(
                src_ref=stage_m.at[s],
                dst_ref=comm_m.at[s],
                send_sem=send_m.at[s],
                recv_sem=recv_m.at[s],
                device_id=(left,),
                device_id_type=pl.DeviceIdType.MESH,
            )
            return rp, rm

        def ag_copies(s):
            rp = pltpu.make_async_remote_copy(
                src_ref=ag_p.at[s],
                dst_ref=ag_p.at[s + 1],
                send_sem=send_p.at[RS_HOPS + s],
                recv_sem=recv_p.at[RS_HOPS + s],
                device_id=(right,),
                device_id_type=pl.DeviceIdType.MESH,
            )
            rm = pltpu.make_async_remote_copy(
                src_ref=ag_m.at[s],
                dst_ref=ag_m.at[s + 1],
                send_sem=send_m.at[RS_HOPS + s],
                recv_sem=recv_m.at[RS_HOPS + s],
                device_id=(left,),
                device_id_type=pl.DeviceIdType.MESH,
            )
            return rp, rm

        accA = compute_half(offA(r))
        stage_p[0, :, :] = accA.astype(jnp.bfloat16)
        rp, rm = rs_copies(0)
        rp.start()
        out_ref[pl.ds(offA(r), HALF), :] = accA
        accB = compute_half(offB(r))
        stage_m[0, :, :] = accB.astype(jnp.bfloat16)
        rm.start()
        out_ref[pl.ds(offB(r), HALF), :] = accB

        for s in range(RS_HOPS):
            oa = offA(r - s - 1)
            out_ref[pl.ds(oa, HALF), :] = compute_half(oa)
            ob = offB(r + s + 1)
            out_ref[pl.ds(ob, HALF), :] = compute_half(ob)
            rp.wait()
            rm.wait()
            ap = _perm(_mod(r - s - 1, N_DEV))
            am = _perm(_mod(r + s + 1, N_DEV))
            slp = pl.ds(ap * CHUNK, HALF)
            slm = pl.ds(am * CHUNK + HALF, HALF)
            vp = out_ref[slp, :] + comm_p[s, :, :].astype(jnp.float32)
            vm = out_ref[slm, :] + comm_m[s, :, :].astype(jnp.float32)
            if s < RS_HOPS - 1:
                stage_p[s + 1, :, :] = vp.astype(jnp.bfloat16)
                stage_m[s + 1, :, :] = vm.astype(jnp.bfloat16)
                rp, rm = rs_copies(s + 1)
            else:
                ag_p[0, :, :] = vp.astype(jnp.bfloat16)
                ag_m[0, :, :] = vm.astype(jnp.bfloat16)
                rp, rm = ag_copies(0)
            rp.start()
            rm.start()
            out_ref[slp, :] = vp
            out_ref[slm, :] = vm

        for s in range(AG_HOPS):
            rp.wait()
            rm.wait()
            if s < AG_HOPS - 1:
                rp, rm = ag_copies(s + 1)
                rp.start()
                rm.start()
            cp = _perm(_mod(r - s, N_DEV))
            cm = _perm(_mod(r + s, N_DEV))
            out_ref[pl.ds(cp * CHUNK, HALF), :] = ag_p[s + 1, :, :].astype(jnp.float32)
            out_ref[pl.ds(cm * CHUNK + HALF, HALF), :] = ag_m[s + 1, :, :].astype(jnp.float32)

    return pl.pallas_call(
        body,
        out_shape=jax.ShapeDtypeStruct((N_TOK, D_HID), jnp.float32),
        in_specs=[
            pl.BlockSpec(memory_space=pltpu.VMEM),
            pl.BlockSpec(memory_space=pltpu.VMEM),
            pl.BlockSpec(memory_space=pltpu.VMEM),
            pl.BlockSpec(memory_space=pltpu.VMEM),
        ],
        out_specs=pl.BlockSpec(memory_space=pltpu.VMEM),
        scratch_shapes=[
            pltpu.VMEM((N_TOK, 2), jnp.float32),
            pltpu.VMEM((RS_HOPS, HALF, D_HID), jnp.bfloat16),
            pltpu.VMEM((RS_HOPS, HALF, D_HID), jnp.bfloat16),
            pltpu.VMEM((RS_HOPS, HALF, D_HID), jnp.bfloat16),
            pltpu.VMEM((RS_HOPS, HALF, D_HID), jnp.bfloat16),
            pltpu.VMEM((N_DEV, HALF, D_HID), jnp.bfloat16),
            pltpu.VMEM((N_DEV, HALF, D_HID), jnp.bfloat16),
            pltpu.SemaphoreType.DMA((N_SEM,)),
            pltpu.SemaphoreType.DMA((N_SEM,)),
            pltpu.SemaphoreType.DMA((N_SEM,)),
            pltpu.SemaphoreType.DMA((N_SEM,)),
        ],
        compiler_params=pltpu.CompilerParams(collective_id=0),
    )(x, router_W, route_idx, expert_W)


# device time: 81785 ns/iter; 1.6653x vs baseline; 1.0217x over previous
import jax
import jax.numpy as jnp
from jax import lax
from jax.experimental import pallas as pl
from jax.experimental.pallas import tpu as pltpu

N_DEV = 8
N_TOK = 2048
D_MODEL = 512
D_HID = 1024
N_EXP = 32
EXP_PER_DEV = N_EXP // N_DEV
CHUNK = N_TOK // N_DEV
HALF = CHUNK // 2
RS_HOPS = N_DEV - 1
AG_HOPS = N_DEV - 1
N_SEM = RS_HOPS + AG_HOPS


def _mod(v, n):
    return lax.rem(v + 4 * n, n)


def kernel(x, router_W, route_idx, expert_W):
    def body(x_ref, rw_ref, idx_ref, ew_ref, out_ref,
             comm_p, comm_m, stage_p, stage_m, ag_p, ag_m,
             send_p, recv_p, send_m, recv_m):
        my = lax.axis_index("i")
        def _perm(q):
            return jnp.where(q < 4, q, 11 - q)

        r = _perm(my)
        left = _perm(_mod(r - 1, N_DEV))
        right = _perm(_mod(r + 1, N_DEV))

        barrier = pltpu.get_barrier_semaphore()
        for nbr in (left, right):
            pl.semaphore_signal(barrier, inc=1, device_id=(nbr,),
                                device_id_type=pl.DeviceIdType.MESH)
        pl.semaphore_wait(barrier, 2)

        base = my * EXP_PER_DEV

        def compute_half(off):
            xc = x_ref[pl.ds(off, HALF), :]
            e0c = idx_ref[pl.ds(off, HALF), 0:1]
            e1c = idx_ref[pl.ds(off, HALF), 1:2]
            sc = jnp.dot(xc, rw_ref[:, :], preferred_element_type=jnp.float32)
            probs = jnp.exp(sc - jnp.max(sc, axis=-1, keepdims=True))
            colc = lax.broadcasted_iota(jnp.int32, (HALF, N_EXP), 1)
            g0c = jnp.sum(jnp.where(colc == e0c, probs, 0.0),
                          axis=-1, keepdims=True)
            g1c = jnp.sum(jnp.where(colc == e1c, probs, 0.0),
                          axis=-1, keepdims=True)
            gsum = g0c + g1c
            g0c = g0c / gsum
            g1c = g1c / gsum
            acc = jnp.zeros((HALF, D_HID), jnp.float32)
            for k in range(EXP_PER_DEV):
                ge = base + k
                gate = (jnp.where(e0c == ge, g0c, 0.0)
                        + jnp.where(e1c == ge, g1c, 0.0))
                acc = acc + jnp.dot(xc * gate, ew_ref[k],
                                    preferred_element_type=jnp.float32)
            return acc

        def offA(slot):
            return _perm(_mod(slot, N_DEV)) * CHUNK

        def offB(slot):
            return _perm(_mod(slot, N_DEV)) * CHUNK + HALF

        def rs_copies(s):
            rp = pltpu.make_async_remote_copy(
                src_ref=stage_p.at[s],
                dst_ref=comm_p.at[s],
                send_sem=send_p.at[s],
                recv_sem=recv_p.at[s],
                device_id=(right,),
                device_id_type=pl.DeviceIdType.MESH,
            )
            rm = pltpu.make_async_remote_copy(
                src_ref=stage_m.at[s],
                dst_ref=comm_m.at[s],
                send_sem=send_m.at[s],
                recv_sem=recv_m.at[s],
                device_id=(left,),
                device_id_type=pl.DeviceIdType.MESH,
            )
            return rp, rm

        def ag_copies(s):
            rp = pltpu.make_async_remote_copy(
                src_ref=ag_p.at[s],
                dst_ref=ag_p.at[s + 1],
                send_sem=send_p.at[RS_HOPS + s],
                recv_sem=recv_p.at[RS_HOPS + s],
                device_id=(right,),
                device_id_type=pl.DeviceIdType.MESH,
            )
            rm = pltpu.make_async_remote_copy(
                src_ref=ag_m.at[s],
                dst_ref=ag_m.at[s + 1],
                send_sem=send_m.at[RS_HOPS + s],
                recv_sem=recv_m.at[RS_HOPS + s],
                device_id=(left,),
                device_id_type=pl.DeviceIdType.MESH,
            )
            return rp, rm

        accA = compute_half(offA(r))
        stage_p[0, :, :] = accA.astype(jnp.bfloat16)
        rp, rm = rs_copies(0)
        rp.start()
        out_ref[pl.ds(offA(r), HALF), :] = accA
        accB = compute_half(offB(r))
        stage_m[0, :, :] = accB.astype(jnp.bfloat16)
        rm.start()
        out_ref[pl.ds(offB(r), HALF), :] = accB

        for s in range(RS_HOPS):
            oa = offA(r - s - 1)
            out_ref[pl.ds(oa, HALF), :] = compute_half(oa)
            ob = offB(r + s + 1)
            out_ref[pl.ds(ob, HALF), :] = compute_half(ob)
            rp.wait()
            rm.wait()
            ap = _perm(_mod(r - s - 1, N_DEV))
            am = _perm(_mod(r + s + 1, N_DEV))
            slp = pl.ds(ap * CHUNK, HALF)
            slm = pl.ds(am * CHUNK + HALF, HALF)
            vp = out_ref[slp, :] + comm_p[s, :, :].astype(jnp.float32)
            vm = out_ref[slm, :] + comm_m[s, :, :].astype(jnp.float32)
            if s < RS_HOPS - 1:
                stage_p[s + 1, :, :] = vp.astype(jnp.bfloat16)
                stage_m[s + 1, :, :] = vm.astype(jnp.bfloat16)
                rp, rm = rs_copies(s + 1)
            else:
                ag_p[0, :, :] = vp.astype(jnp.bfloat16)
                ag_m[0, :, :] = vm.astype(jnp.bfloat16)
                rp, rm = ag_copies(0)
            rp.start()
            rm.start()
            out_ref[slp, :] = vp
            out_ref[slm, :] = vm

        for s in range(AG_HOPS):
            rp.wait()
            rm.wait()
            if s < AG_HOPS - 1:
                rp, rm = ag_copies(s + 1)
                rp.start()
                rm.start()
            cp = _perm(_mod(r - s, N_DEV))
            cm = _perm(_mod(r + s, N_DEV))
            out_ref[pl.ds(cp * CHUNK, HALF), :] = ag_p[s + 1, :, :].astype(jnp.float32)
            out_ref[pl.ds(cm * CHUNK + HALF, HALF), :] = ag_m[s + 1, :, :].astype(jnp.float32)

    return pl.pallas_call(
        body,
        out_shape=jax.ShapeDtypeStruct((N_TOK, D_HID), jnp.float32),
        in_specs=[
            pl.BlockSpec(memory_space=pltpu.VMEM),
            pl.BlockSpec(memory_space=pltpu.VMEM),
            pl.BlockSpec(memory_space=pltpu.VMEM),
            pl.BlockSpec(memory_space=pltpu.VMEM),
        ],
        out_specs=pl.BlockSpec(memory_space=pltpu.VMEM),
        scratch_shapes=[
            pltpu.VMEM((RS_HOPS, HALF, D_HID), jnp.bfloat16),
            pltpu.VMEM((RS_HOPS, HALF, D_HID), jnp.bfloat16),
            pltpu.VMEM((RS_HOPS, HALF, D_HID), jnp.bfloat16),
            pltpu.VMEM((RS_HOPS, HALF, D_HID), jnp.bfloat16),
            pltpu.VMEM((N_DEV, HALF, D_HID), jnp.bfloat16),
            pltpu.VMEM((N_DEV, HALF, D_HID), jnp.bfloat16),
            pltpu.SemaphoreType.DMA((N_SEM,)),
            pltpu.SemaphoreType.DMA((N_SEM,)),
            pltpu.SemaphoreType.DMA((N_SEM,)),
            pltpu.SemaphoreType.DMA((N_SEM,)),
        ],
        compiler_params=pltpu.CompilerParams(collective_id=0),
    )(x, router_W, route_idx, expert_W)


# device time: 81063 ns/iter; 1.6802x vs baseline; 1.0089x over previous
import jax
import jax.numpy as jnp
from jax import lax
from jax.experimental import pallas as pl
from jax.experimental.pallas import tpu as pltpu

N_DEV = 8
N_TOK = 2048
D_MODEL = 512
D_HID = 1024
N_EXP = 32
EXP_PER_DEV = N_EXP // N_DEV
CHUNK = N_TOK // N_DEV
HALF = CHUNK // 2
RS_HOPS = N_DEV - 1
AG_HOPS = N_DEV - 1
N_SEM = RS_HOPS + AG_HOPS


def _mod(v, n):
    return lax.rem(v + 4 * n, n)


def kernel(x, router_W, route_idx, expert_W):
    def body(x_ref, rw_ref, idx_ref, ew_ref, out_ref,
             comm_p, comm_m, stage_p, stage_m, ag_p, ag_m,
             send_p, recv_p, send_m, recv_m):
        my = lax.axis_index("i")
        def _perm(q):
            return jnp.where(q < 4, q, 11 - q)

        r = _perm(my)
        left = _perm(_mod(r - 1, N_DEV))
        right = _perm(_mod(r + 1, N_DEV))

        barrier = pltpu.get_barrier_semaphore()
        for nbr in (left, right):
            pl.semaphore_signal(barrier, inc=1, device_id=(nbr,),
                                device_id_type=pl.DeviceIdType.MESH)
        pl.semaphore_wait(barrier, 2)

        base = my * EXP_PER_DEV

        def compute_half(off):
            xc = x_ref[pl.ds(off, HALF), :]
            e0c = idx_ref[pl.ds(off, HALF), 0:1]
            e1c = idx_ref[pl.ds(off, HALF), 1:2]
            sc = jnp.dot(xc, rw_ref[:, :], preferred_element_type=jnp.float32)
            probs = jnp.exp(sc - jnp.max(sc, axis=-1, keepdims=True))
            colc = lax.broadcasted_iota(jnp.int32, (HALF, N_EXP), 1)
            g0c = jnp.sum(jnp.where(colc == e0c, probs, 0.0),
                          axis=-1, keepdims=True)
            g1c = jnp.sum(jnp.where(colc == e1c, probs, 0.0),
                          axis=-1, keepdims=True)
            gsum = g0c + g1c
            g0c = g0c / gsum
            g1c = g1c / gsum
            acc = jnp.zeros((HALF, D_HID), jnp.float32)
            for k in range(EXP_PER_DEV):
                ge = base + k
                gate = (jnp.where(e0c == ge, g0c, 0.0)
                        + jnp.where(e1c == ge, g1c, 0.0))
                acc = acc + jnp.dot(xc * gate, ew_ref[k],
                                    preferred_element_type=jnp.float32)
            return acc

        def offA(slot):
            return _perm(_mod(slot, N_DEV)) * CHUNK

        def offB(slot):
            return _perm(_mod(slot, N_DEV)) * CHUNK + HALF

        def rs_copy_p(s):
            return pltpu.make_async_remote_copy(
                src_ref=stage_p.at[s],
                dst_ref=comm_p.at[s],
                send_sem=send_p.at[s],
                recv_sem=recv_p.at[s],
                device_id=(right,),
                device_id_type=pl.DeviceIdType.MESH,
            )

        def rs_copy_m(s):
            return pltpu.make_async_remote_copy(
                src_ref=stage_m.at[s],
                dst_ref=comm_m.at[s],
                send_sem=send_m.at[s],
                recv_sem=recv_m.at[s],
                device_id=(left,),
                device_id_type=pl.DeviceIdType.MESH,
            )

        def ag_copy_p(s):
            return pltpu.make_async_remote_copy(
                src_ref=ag_p.at[s],
                dst_ref=ag_p.at[s + 1],
                send_sem=send_p.at[RS_HOPS + s],
                recv_sem=recv_p.at[RS_HOPS + s],
                device_id=(right,),
                device_id_type=pl.DeviceIdType.MESH,
            )

        def ag_copy_m(s):
            return pltpu.make_async_remote_copy(
                src_ref=ag_m.at[s],
                dst_ref=ag_m.at[s + 1],
                send_sem=send_m.at[RS_HOPS + s],
                recv_sem=recv_m.at[RS_HOPS + s],
                device_id=(left,),
                device_id_type=pl.DeviceIdType.MESH,
            )

        accA = compute_half(offA(r))
        stage_p[0, :, :] = accA.astype(jnp.bfloat16)
        rp = rs_copy_p(0)
        rp.start()
        out_ref[pl.ds(offA(r), HALF), :] = accA
        accB = compute_half(offB(r))
        stage_m[0, :, :] = accB.astype(jnp.bfloat16)
        rm = rs_copy_m(0)
        rm.start()
        out_ref[pl.ds(offB(r), HALF), :] = accB

        for s in range(RS_HOPS):
            oa = offA(r - s - 1)
            out_ref[pl.ds(oa, HALF), :] = compute_half(oa)
            ob = offB(r + s + 1)
            out_ref[pl.ds(ob, HALF), :] = compute_half(ob)
            ap = _perm(_mod(r - s - 1, N_DEV))
            am = _perm(_mod(r + s + 1, N_DEV))
            slp = pl.ds(ap * CHUNK, HALF)
            slm = pl.ds(am * CHUNK + HALF, HALF)
            rp.wait()
            vp = out_ref[slp, :] + comm_p[s, :, :].astype(jnp.float32)
            if s < RS_HOPS - 1:
                stage_p[s + 1, :, :] = vp.astype(jnp.bfloat16)
                rp = rs_copy_p(s + 1)
            else:
                ag_p[0, :, :] = vp.astype(jnp.bfloat16)
                rp = ag_copy_p(0)
            rp.start()
            out_ref[slp, :] = vp
            rm.wait()
            vm = out_ref[slm, :] + comm_m[s, :, :].astype(jnp.float32)
            if s < RS_HOPS - 1:
                stage_m[s + 1, :, :] = vm.astype(jnp.bfloat16)
                rm = rs_copy_m(s + 1)
            else:
                ag_m[0, :, :] = vm.astype(jnp.bfloat16)
                rm = ag_copy_m(0)
            rm.start()
            out_ref[slm, :] = vm

        for s in range(AG_HOPS):
            rp.wait()
            if s < AG_HOPS - 1:
                rp = ag_copy_p(s + 1)
                rp.start()
            cp = _perm(_mod(r - s, N_DEV))
            out_ref[pl.ds(cp * CHUNK, HALF), :] = ag_p[s + 1, :, :].astype(jnp.float32)
            rm.wait()
            if s < AG_HOPS - 1:
                rm = ag_copy_m(s + 1)
                rm.start()
            cm = _perm(_mod(r + s, N_DEV))
            out_ref[pl.ds(cm * CHUNK + HALF, HALF), :] = ag_m[s + 1, :, :].astype(jnp.float32)

    return pl.pallas_call(
        body,
        out_shape=jax.ShapeDtypeStruct((N_TOK, D_HID), jnp.float32),
        in_specs=[
            pl.BlockSpec(memory_space=pltpu.VMEM),
            pl.BlockSpec(memory_space=pltpu.VMEM),
            pl.BlockSpec(memory_space=pltpu.VMEM),
            pl.BlockSpec(memory_space=pltpu.VMEM),
        ],
        out_specs=pl.BlockSpec(memory_space=pltpu.VMEM),
        scratch_shapes=[
            pltpu.VMEM((RS_HOPS, HALF, D_HID), jnp.bfloat16),
            pltpu.VMEM((RS_HOPS, HALF, D_HID), jnp.bfloat16),
            pltpu.VMEM((RS_HOPS, HALF, D_HID), jnp.bfloat16),
            pltpu.VMEM((RS_HOPS, HALF, D_HID), jnp.bfloat16),
            pltpu.VMEM((N_DEV, HALF, D_HID), jnp.bfloat16),
            pltpu.VMEM((N_DEV, HALF, D_HID), jnp.bfloat16),
            pltpu.SemaphoreType.DMA((N_SEM,)),
            pltpu.SemaphoreType.DMA((N_SEM,)),
            pltpu.SemaphoreType.DMA((N_SEM,)),
            pltpu.SemaphoreType.DMA((N_SEM,)),
        ],
        compiler_params=pltpu.CompilerParams(collective_id=0),
    )(x, router_W, route_idx, expert_W)
